# baseline (speedup 1.0000x reference)
import functools

import jax
import jax.numpy as jnp
from jax import lax
from jax.experimental import pallas as pl
from jax.experimental.pallas import tpu as pltpu
from jax.experimental.pallas import tpu_sc as plsc


def _make_sc_gather(n, V, D, NW, NC):
    per_w = n // NW
    G = 128
    ng = per_w // G

    mesh = plsc.VectorSubcoreMesh(core_axis_name="c", subcore_axis_name="s")

    @functools.partial(
        pl.kernel,
        mesh=mesh,
        out_type=jax.ShapeDtypeStruct((n, D), jnp.float32),
        compiler_params=pltpu.CompilerParams(use_tc_tiling_on_sc=False),
        scratch_types=[
            pltpu.VMEM((ng, G), jnp.int32),
            pltpu.VMEM((2, G, D), jnp.float32),
            pltpu.SemaphoreType.DMA,
            pltpu.SemaphoreType.DMA,
        ],
    )
    def k(items_hbm, table_hbm, out_hbm, idx_v, rows_v, gsem, ssem):
        wid = lax.axis_index("s") * NC + lax.axis_index("c")
        base = pl.multiple_of(wid * per_w, per_w)
        pltpu.sync_copy(items_hbm.at[wid], idx_v)

        def gather_desc(j, slot):
            return pltpu.make_async_copy(
                table_hbm.at[idx_v.at[j]], rows_v.at[slot], gsem)

        def scat_desc(j, slot):
            off = pl.multiple_of(base + j * G, G)
            return pltpu.make_async_copy(
                rows_v.at[slot], out_hbm.at[pl.ds(off, G)], ssem)

        gather_desc(0, 0).start()

        def step(j, slot):
            gather_desc(j, slot).wait()

            @pl.when(j >= 1)
            def _():
                scat_desc(j - 1, 1 - slot).wait()

            @pl.when(j + 1 < ng)
            def _():
                gather_desc(j + 1, 1 - slot).start()

            scat_desc(j, slot).start()

        def outer(i, carry):
            g = i * 2
            step(g, 0)
            step(g + 1, 1)
            return carry

        lax.fori_loop(0, ng // 2, outer, 0)
        scat_desc(ng - 1, 1).wait()

    return k


def kernel(items, table):
    B, T = items.shape
    V, D = table.shape
    n = B * T
    NC, NS = 2, 16
    NW = NC * NS
    per_w = n // NW
    G = 128
    assert n % (NW * G) == 0 and (per_w // G) % 2 == 0

    idx = items.reshape(NW, per_w // G, G).astype(jnp.int32)
    out = _make_sc_gather(n, V, D, NW, NC)(idx, table)
    return out.reshape(B, T, D)


# speedup vs baseline: 1.0739x; 1.0739x over previous
import functools

import jax
import jax.numpy as jnp
from jax import lax
from jax.experimental import pallas as pl
from jax.experimental.pallas import tpu as pltpu
from jax.experimental.pallas import tpu_sc as plsc


def _make_sc_gather(n, V, D, NW, NC):
    per_w = n // NW
    G = 128
    ng = per_w // G
    NBUF = 8
    LAG = 4

    mesh = plsc.VectorSubcoreMesh(core_axis_name="c", subcore_axis_name="s")

    @functools.partial(
        pl.kernel,
        mesh=mesh,
        out_type=jax.ShapeDtypeStruct((n, D), jnp.float32),
        compiler_params=pltpu.CompilerParams(use_tc_tiling_on_sc=False),
        scratch_types=[
            pltpu.VMEM((ng, G), jnp.int32),
            pltpu.VMEM((NBUF, G, D), jnp.float32),
            pltpu.SemaphoreType.DMA,
            pltpu.SemaphoreType.DMA,
        ],
    )
    def k(items_hbm, table_hbm, out_hbm, idx_v, rows_v, gsem, ssem):
        wid = lax.axis_index("s") * NC + lax.axis_index("c")
        base = pl.multiple_of(wid * per_w, per_w)
        pltpu.sync_copy(items_hbm.at[wid], idx_v)

        def gather_desc(j, slot):
            return pltpu.make_async_copy(
                table_hbm.at[idx_v.at[j]], rows_v.at[slot], gsem)

        def scat_desc(j, slot):
            off = pl.multiple_of(base + j * G, G)
            return pltpu.make_async_copy(
                rows_v.at[slot], out_hbm.at[pl.ds(off, G)], ssem)

        for b in range(LAG):
            gather_desc(b, b).start()

        def step(j, slot):
            @pl.when(j >= LAG)
            def _():
                scat_desc(j - LAG, (slot + LAG) % NBUF).wait()

            @pl.when(j + LAG < ng)
            def _():
                gather_desc(j + LAG, (slot + LAG) % NBUF).start()

            gather_desc(j, slot).wait()
            scat_desc(j, slot).start()

        def outer(i, carry):
            g = i * NBUF
            for b in range(NBUF):
                step(g + b, b)
            return carry

        lax.fori_loop(0, ng // NBUF, outer, 0)
        for j in range(ng - LAG, ng):
            scat_desc(j, j % NBUF).wait()

    return k


def kernel(items, table):
    B, T = items.shape
    V, D = table.shape
    n = B * T
    NC, NS = 2, 16
    NW = NC * NS
    per_w = n // NW
    G = 128
    assert n % (NW * G) == 0 and (per_w // G) % 8 == 0

    idx = items.reshape(NW, per_w // G, G).astype(jnp.int32)
    out = _make_sc_gather(n, V, D, NW, NC)(idx, table)
    return out.reshape(B, T, D)


# speedup vs baseline: 1.3079x; 1.2178x over previous
import functools

import jax
import jax.numpy as jnp
from jax import lax
from jax.experimental import pallas as pl
from jax.experimental.pallas import tpu as pltpu
from jax.experimental.pallas import tpu_sc as plsc


def _make_sc_gather(n, V, DP, NW, NC):
    per_w = n // NW
    G = 128
    ng = per_w // G
    NBUF = 4
    LAG = 2

    mesh = plsc.VectorSubcoreMesh(core_axis_name="c", subcore_axis_name="s")

    @functools.partial(
        pl.kernel,
        mesh=mesh,
        out_type=jax.ShapeDtypeStruct((n, DP), jnp.float32),
        compiler_params=pltpu.CompilerParams(use_tc_tiling_on_sc=True),
        scratch_types=[
            pltpu.VMEM((ng, G), jnp.int32),
            pltpu.VMEM((NBUF, G, DP), jnp.float32),
            pltpu.SemaphoreType.DMA,
            pltpu.SemaphoreType.DMA,
        ],
    )
    def k(items_hbm, table_hbm, out_hbm, idx_v, rows_v, gsem, ssem):
        wid = lax.axis_index("s") * NC + lax.axis_index("c")
        base = pl.multiple_of(wid * per_w, per_w)
        pltpu.sync_copy(items_hbm.at[wid], idx_v)

        def gather_desc(j, slot):
            return pltpu.make_async_copy(
                table_hbm.at[idx_v.at[j]], rows_v.at[slot], gsem)

        def scat_desc(j, slot):
            off = pl.multiple_of(base + j * G, G)
            return pltpu.make_async_copy(
                rows_v.at[slot], out_hbm.at[pl.ds(off, G)], ssem)

        for b in range(LAG):
            gather_desc(b, b).start()

        def step(j, slot):
            @pl.when(j >= LAG)
            def _():
                scat_desc(j - LAG, (slot + LAG) % NBUF).wait()

            @pl.when(j + LAG < ng)
            def _():
                gather_desc(j + LAG, (slot + LAG) % NBUF).start()

            gather_desc(j, slot).wait()
            scat_desc(j, slot).start()

        def outer(i, carry):
            g = i * NBUF
            for b in range(NBUF):
                step(g + b, b)
            return carry

        lax.fori_loop(0, ng // NBUF, outer, 0)
        for j in range(ng - LAG, ng):
            scat_desc(j, j % NBUF).wait()

    return k


def kernel(items, table):
    B, T = items.shape
    V, D = table.shape
    DP = 128
    n = B * T
    NC, NS = 2, 16
    NW = NC * NS
    per_w = n // NW
    G = 128
    assert n % (NW * G) == 0 and (per_w // G) % 4 == 0

    table_p = jnp.pad(table, ((0, 0), (0, DP - D)))
    idx = items.reshape(NW, per_w // G, G).astype(jnp.int32)
    out = _make_sc_gather(n, V, DP, NW, NC)(idx, table_p)
    return out[:, :D].reshape(B, T, D)
